# baseline (device time: 393168 ns/iter reference)
import jax
import jax.numpy as jnp
import numpy as np
from jax import lax
from jax.experimental import pallas as pl
from jax.experimental.pallas import tpu as pltpu

N_DEV = 32

_CYCLE = np.array(
    [1, 2, 5, 6, 14, 13, 10, 9, 17, 18, 21, 22, 30, 29, 26, 25,
     24, 27, 28, 31, 23, 20, 19, 16, 8, 11, 12, 15, 7, 4, 3, 0],
    dtype=np.int32,
)
_CYCLE_INV = np.argsort(_CYCLE).astype(np.int32)


def kernel(x, w_mat, scale_x, scale_w):
    m, k = x.shape
    _, n = w_mat.shape
    ch = m // N_DEV
    n4 = n // 4

    def body(x_ref, w_ref, sx_ref, sw_ref, ids_ref, out_ref,
             stage, rs_comm, ag_comm, send_sems, recv_sems, credits):
        rho = ids_ref[0]
        left = ids_ref[1]
        right = ids_ref[2]

        rings = (
            (0, right, left, -1, 0 * n4),
            (1, right, left, -1, 1 * n4),
            (2, left, right, 1, 2 * n4),
            (3, left, right, 1, 3 * n4),
        )

        def desc(r, sdev, src, dst, par):
            return pltpu.make_async_remote_copy(
                src_ref=src, dst_ref=dst,
                send_sem=send_sems.at[r, par], recv_sem=recv_sems.at[r, par],
                device_id=(sdev,), device_id_type=pl.DeviceIdType.MESH,
            )

        wb = w_ref[...].astype(jnp.bfloat16)

        def gemm_chunk(c, _):
            out_ref[pl.ds(c * ch, ch), :] = jnp.dot(
                x_ref[pl.ds(c * ch, ch), :].astype(jnp.bfloat16),
                wb,
                preferred_element_type=jnp.float32,
            )
            return _

        lax.fori_loop(0, N_DEV, gemm_chunk, None)

        bar = pltpu.get_barrier_semaphore()
        for nbr in (left, right):
            pl.semaphore_signal(
                bar, inc=1, device_id=(nbr,),
                device_id_type=pl.DeviceIdType.MESH,
            )
        pl.semaphore_wait(bar, 2)

        for r, sdev, cdev, sign, c0 in rings:
            stage[r, 0] = out_ref[
                pl.ds(rho * ch, ch), c0:c0 + n4].astype(jnp.bfloat16)
            desc(r, sdev, stage.at[r, 0], rs_comm.at[r, 0], 0).start()

        def rs_hop(s, _):
            par = jnp.mod(s, 2)
            nxt = jnp.mod(s + 1, 2)
            for r, sdev, cdev, sign, c0 in rings:
                ridx = jnp.mod(rho + sign * (s + 1), N_DEV)
                desc(r, sdev, stage.at[r, par], rs_comm.at[r, par],
                     par).wait_recv()
                summ = (out_ref[pl.ds(ridx * ch, ch), c0:c0 + n4]
                        + rs_comm[r, par].astype(jnp.float32))
                out_ref[pl.ds(ridx * ch, ch), c0:c0 + n4] = summ

                @pl.when(s < N_DEV - 2)
                def _(r=r, sdev=sdev, summ=summ, nxt=nxt, s=s):
                    @pl.when(s >= 1)
                    def _():
                        desc(r, sdev, stage.at[r, nxt], rs_comm.at[r, nxt],
                             nxt).wait_send()
                        pl.semaphore_wait(credits.at[r], 1)
                    stage[r, nxt] = summ.astype(jnp.bfloat16)
                    desc(r, sdev, stage.at[r, nxt], rs_comm.at[r, nxt],
                         nxt).start()

                @pl.when(s < N_DEV - 3)
                def _(r=r, cdev=cdev):
                    pl.semaphore_signal(
                        credits.at[r], inc=1, device_id=(cdev,),
                        device_id_type=pl.DeviceIdType.MESH,
                    )
            return _

        lax.fori_loop(0, N_DEV - 1, rs_hop, None)

        for r, sdev, cdev, sign, c0 in rings:
            desc(r, sdev, stage.at[r, 1], rs_comm.at[r, 1], 1).wait_send()
            desc(r, sdev, stage.at[r, 0], rs_comm.at[r, 0], 0).wait_send()

        scale = sx_ref[0] * sw_ref[0]
        for r, sdev, cdev, sign, c0 in rings:
            own = jnp.mod(rho - sign, N_DEV)
            rows = pl.ds(own * ch, ch)
            out_ref[rows, c0:c0 + n4] = jnp.maximum(
                out_ref[rows, c0:c0 + n4] * scale, 0.0)
            stage[r, 0] = out_ref[rows, c0:c0 + n4].astype(jnp.bfloat16)

        for r, sdev, cdev, sign, c0 in rings:
            desc(r, sdev, stage.at[r, 0], ag_comm.at[r, 0], 0).start()

        def ag_hop(s, _):
            par = jnp.mod(s, 2)
            nxt = jnp.mod(s + 1, 2)
            for r, sdev, cdev, sign, c0 in rings:
                ridx = jnp.mod(rho + sign * s, N_DEV)
                desc(r, sdev, ag_comm.at[r, par], ag_comm.at[r, par],
                     par).wait_recv()

                @pl.when(s < N_DEV - 2)
                def _(r=r, sdev=sdev, par=par, nxt=nxt, s=s):
                    @pl.when(s >= 1)
                    def _():
                        pl.semaphore_wait(credits.at[r], 1)
                    desc(r, sdev, ag_comm.at[r, par], ag_comm.at[r, nxt],
                         nxt).start()

                out_ref[pl.ds(ridx * ch, ch), c0:c0 + n4] = (
                    ag_comm[r, par].astype(jnp.float32))

                @pl.when(s < N_DEV - 2)
                def _(r=r, sdev=sdev, par=par, nxt=nxt, s=s):
                    @pl.when(s == 0)
                    def _():
                        desc(r, sdev, stage.at[r, 0], ag_comm.at[r, 0],
                             0).wait_send()
                    desc(r, sdev, ag_comm.at[r, par], ag_comm.at[r, nxt],
                         nxt).wait_send()

                @pl.when(s < N_DEV - 3)
                def _(r=r, cdev=cdev):
                    pl.semaphore_signal(
                        credits.at[r], inc=1, device_id=(cdev,),
                        device_id_type=pl.DeviceIdType.MESH,
                    )
            return _

        lax.fori_loop(0, N_DEV - 1, ag_hop, None)

    mesh_idx = lax.axis_index("i")
    rho = jnp.take(jnp.asarray(_CYCLE_INV), mesh_idx)
    cyc = jnp.asarray(_CYCLE)
    left = jnp.take(cyc, jnp.mod(rho - 1, N_DEV))
    right = jnp.take(cyc, jnp.mod(rho + 1, N_DEV))
    ids = jnp.stack([rho, left, right]).astype(jnp.int32)

    return pl.pallas_call(
        body,
        out_shape=jax.ShapeDtypeStruct((m, n), jnp.float32),
        in_specs=[
            pl.BlockSpec(memory_space=pltpu.VMEM),
            pl.BlockSpec(memory_space=pltpu.VMEM),
            pl.BlockSpec(memory_space=pltpu.SMEM),
            pl.BlockSpec(memory_space=pltpu.SMEM),
            pl.BlockSpec(memory_space=pltpu.SMEM),
        ],
        out_specs=pl.BlockSpec(memory_space=pltpu.VMEM),
        scratch_shapes=[
            pltpu.VMEM((4, 2, ch, n4), jnp.bfloat16),
            pltpu.VMEM((4, 2, ch, n4), jnp.bfloat16),
            pltpu.VMEM((4, 2, ch, n4), jnp.bfloat16),
            pltpu.SemaphoreType.DMA((4, 2)),
            pltpu.SemaphoreType.DMA((4, 2)),
            pltpu.SemaphoreType.REGULAR((4,)),
        ],
        compiler_params=pltpu.CompilerParams(
            collective_id=0,
            vmem_limit_bytes=60 * 1024 * 1024,
        ),
    )(x, w_mat, scale_x, scale_w, ids)


# device time: 359484 ns/iter; 1.0937x vs baseline; 1.0937x over previous
import jax
import jax.numpy as jnp
import numpy as np
from jax import lax
from jax.experimental import pallas as pl
from jax.experimental.pallas import tpu as pltpu

N_DEV = 32

_CYCLE = np.array(
    [1, 2, 5, 6, 14, 13, 10, 9, 17, 18, 21, 22, 30, 29, 26, 25,
     24, 27, 28, 31, 23, 20, 19, 16, 8, 11, 12, 15, 7, 4, 3, 0],
    dtype=np.int32,
)
_CYCLE_INV = np.argsort(_CYCLE).astype(np.int32)


def kernel(x, w_mat, scale_x, scale_w):
    m, k = x.shape
    _, n = w_mat.shape
    ch = m // N_DEV

    def body(x_ref, w_ref, sx_ref, sw_ref, ids_ref, out_ref, acc,
             rs_comm_r, rs_comm_l, ag_comm_r, ag_comm_l,
             send_sems, recv_sems, credits):
        rho = ids_ref[0]
        left = ids_ref[1]
        right = ids_ref[2]

        def desc(st, src, dst, par):
            return pltpu.make_async_remote_copy(
                src_ref=src, dst_ref=dst,
                send_sem=send_sems.at[st, par], recv_sem=recv_sems.at[st, par],
                device_id=(right if st == 0 else left,),
                device_id_type=pl.DeviceIdType.MESH,
            )

        def credit_signal(st):
            pl.semaphore_signal(
                credits.at[st], inc=1,
                device_id=(left if st == 0 else right,),
                device_id_type=pl.DeviceIdType.MESH,
            )

        def rows(idx):
            return pl.ds(jnp.mod(idx, N_DEV) * ch, ch)

        wb = w_ref[...].astype(jnp.bfloat16)

        def gemm_chunk(c, _):
            acc[pl.ds(c * ch, ch), :] = jnp.dot(
                x_ref[pl.ds(c * ch, ch), :].astype(jnp.bfloat16),
                wb,
                preferred_element_type=jnp.float32,
            ).astype(jnp.bfloat16)
            return _

        lax.fori_loop(0, N_DEV, gemm_chunk, None)

        bar = pltpu.get_barrier_semaphore()
        for nbr in (left, right):
            pl.semaphore_signal(
                bar, inc=1, device_id=(nbr,),
                device_id_type=pl.DeviceIdType.MESH,
            )
        pl.semaphore_wait(bar, 2)

        desc(0, acc.at[rows(rho + 16), :], rs_comm_r.at[0], 0).start()
        desc(1, acc.at[rows(rho - 15), :], rs_comm_l.at[0], 0).start()

        def rs_hop(s, _):
            par = jnp.mod(s, 2)
            nxt = jnp.mod(s + 1, 2)

            rr = rows(rho + 15 - s)
            desc(0, acc.at[rr, :], rs_comm_r.at[par], par).wait_recv()
            acc[rr, :] = acc[rr, :] + rs_comm_r[par]

            @pl.when(s < 15)
            def _(rr=rr, par=par, nxt=nxt, s=s):
                @pl.when(s >= 1)
                def _():
                    desc(0, acc.at[rr, :], rs_comm_r.at[nxt],
                         nxt).wait_send()
                    pl.semaphore_wait(credits.at[0], 1)
                desc(0, acc.at[rr, :], rs_comm_r.at[nxt], nxt).start()

            @pl.when(s < 14)
            def _(par=par, s=s):
                credit_signal(0)

            @pl.when(s < 15)
            def _(par=par, nxt=nxt, s=s):
                rl = rows(rho - 14 + s)
                desc(1, acc.at[rl, :], rs_comm_l.at[par], par).wait_recv()
                acc[rl, :] = acc[rl, :] + rs_comm_l[par]

                @pl.when(s < 14)
                def _(rl=rl, par=par, nxt=nxt, s=s):
                    @pl.when(s >= 1)
                    def _():
                        desc(1, acc.at[rl, :], rs_comm_l.at[nxt],
                             nxt).wait_send()
                        pl.semaphore_wait(credits.at[1], 1)
                    desc(1, acc.at[rl, :], rs_comm_l.at[nxt], nxt).start()

                @pl.when(s < 13)
                def _(s=s):
                    credit_signal(1)
            return _

        lax.fori_loop(0, 16, rs_hop, None)

        desc(0, acc.at[rows(rho + 1), :], rs_comm_r.at[0], 0).wait_send()
        desc(0, acc.at[rows(rho + 1), :], rs_comm_r.at[1], 1).wait_send()
        desc(1, acc.at[rows(rho - 1), :], rs_comm_l.at[0], 0).wait_send()
        desc(1, acc.at[rows(rho - 1), :], rs_comm_l.at[1], 1).wait_send()

        scale = sx_ref[0] * sw_ref[0]
        own = rows(rho)
        fin = jnp.maximum(acc[own, :].astype(jnp.float32) * scale, 0.0)
        out_ref[own, :] = fin
        acc[own, :] = fin.astype(jnp.bfloat16)

        desc(0, acc.at[own, :], ag_comm_r.at[0], 0).start()
        desc(1, acc.at[own, :], ag_comm_l.at[0], 0).start()

        def ag_hop(s, _):
            par = jnp.mod(s, 2)
            nxt = jnp.mod(s + 1, 2)

            @pl.when(s < 15)
            def _(par=par, nxt=nxt, s=s):
                desc(0, ag_comm_r.at[par], ag_comm_r.at[par],
                     par).wait_recv()

                @pl.when(s < 14)
                def _(par=par, nxt=nxt, s=s):
                    @pl.when(s >= 1)
                    def _():
                        pl.semaphore_wait(credits.at[0], 1)
                    desc(0, ag_comm_r.at[par], ag_comm_r.at[nxt],
                         nxt).start()

                out_ref[rows(rho - 1 - s), :] = (
                    ag_comm_r[par].astype(jnp.float32))

                @pl.when(s < 14)
                def _(par=par, nxt=nxt, s=s):
                    @pl.when(s == 0)
                    def _():
                        desc(0, acc.at[own, :], ag_comm_r.at[0],
                             0).wait_send()
                    desc(0, ag_comm_r.at[par], ag_comm_r.at[nxt],
                         nxt).wait_send()

                @pl.when(s < 13)
                def _(s=s):
                    credit_signal(0)

            desc(1, ag_comm_l.at[par], ag_comm_l.at[par], par).wait_recv()

            @pl.when(s < 15)
            def _(par=par, nxt=nxt, s=s):
                @pl.when(s >= 1)
                def _():
                    pl.semaphore_wait(credits.at[1], 1)
                desc(1, ag_comm_l.at[par], ag_comm_l.at[nxt], nxt).start()

            out_ref[rows(rho + 1 + s), :] = (
                ag_comm_l[par].astype(jnp.float32))

            @pl.when(s < 15)
            def _(par=par, nxt=nxt, s=s):
                @pl.when(s == 0)
                def _():
                    desc(1, acc.at[own, :], ag_comm_l.at[0], 0).wait_send()
                desc(1, ag_comm_l.at[par], ag_comm_l.at[nxt],
                     nxt).wait_send()

            @pl.when(s < 14)
            def _(s=s):
                credit_signal(1)
            return _

        lax.fori_loop(0, 16, ag_hop, None)

    mesh_idx = lax.axis_index("i")
    rho = jnp.take(jnp.asarray(_CYCLE_INV), mesh_idx)
    cyc = jnp.asarray(_CYCLE)
    left = jnp.take(cyc, jnp.mod(rho - 1, N_DEV))
    right = jnp.take(cyc, jnp.mod(rho + 1, N_DEV))
    ids = jnp.stack([rho, left, right]).astype(jnp.int32)

    return pl.pallas_call(
        body,
        out_shape=jax.ShapeDtypeStruct((m, n), jnp.float32),
        in_specs=[
            pl.BlockSpec(memory_space=pltpu.VMEM),
            pl.BlockSpec(memory_space=pltpu.VMEM),
            pl.BlockSpec(memory_space=pltpu.SMEM),
            pl.BlockSpec(memory_space=pltpu.SMEM),
            pl.BlockSpec(memory_space=pltpu.SMEM),
        ],
        out_specs=pl.BlockSpec(memory_space=pltpu.VMEM),
        scratch_shapes=[
            pltpu.VMEM((m, n), jnp.bfloat16),
            pltpu.VMEM((2, ch, n), jnp.bfloat16),
            pltpu.VMEM((2, ch, n), jnp.bfloat16),
            pltpu.VMEM((2, ch, n), jnp.bfloat16),
            pltpu.VMEM((2, ch, n), jnp.bfloat16),
            pltpu.SemaphoreType.DMA((2, 2)),
            pltpu.SemaphoreType.DMA((2, 2)),
            pltpu.SemaphoreType.REGULAR((2,)),
        ],
        compiler_params=pltpu.CompilerParams(
            collective_id=0,
            vmem_limit_bytes=62 * 1024 * 1024,
        ),
    )(x, w_mat, scale_x, scale_w, ids)


# device time: 339750 ns/iter; 1.1572x vs baseline; 1.0581x over previous
import jax
import jax.numpy as jnp
import numpy as np
from jax import lax
from jax.experimental import pallas as pl
from jax.experimental.pallas import tpu as pltpu

N_DEV = 32

_CYCLE = np.array(
    [1, 2, 5, 6, 14, 13, 10, 9, 17, 18, 21, 22, 30, 29, 26, 25,
     24, 27, 28, 31, 23, 20, 19, 16, 8, 11, 12, 15, 7, 4, 3, 0],
    dtype=np.int32,
)
_CYCLE_INV = np.argsort(_CYCLE).astype(np.int32)


def kernel(x, w_mat, scale_x, scale_w):
    m, k = x.shape
    _, n = w_mat.shape
    ch = m // N_DEV
    n2 = n // 2

    def body(x_ref, w_ref, sx_ref, sw_ref, ids_ref, out_ref,
             stage_a, stage_b, rs_comm_a, rs_comm_b, ag_comm_a, ag_comm_b,
             send_sems_a, recv_sems_a, send_sems_b, recv_sems_b,
             credit_a, credit_b):
        rho = ids_ref[0]
        left = ids_ref[1]
        right = ids_ref[2]

        def desc_a(src, dst, par):
            return pltpu.make_async_remote_copy(
                src_ref=src, dst_ref=dst,
                send_sem=send_sems_a.at[par], recv_sem=recv_sems_a.at[par],
                device_id=(right,), device_id_type=pl.DeviceIdType.MESH,
            )

        def desc_b(src, dst, par):
            return pltpu.make_async_remote_copy(
                src_ref=src, dst_ref=dst,
                send_sem=send_sems_b.at[par], recv_sem=recv_sems_b.at[par],
                device_id=(left,), device_id_type=pl.DeviceIdType.MESH,
            )

        wb = w_ref[...].astype(jnp.bfloat16)

        def gemm_chunk(c, _):
            out_ref[pl.ds(c * ch, ch), :] = jnp.dot(
                x_ref[pl.ds(c * ch, ch), :].astype(jnp.bfloat16),
                wb,
                preferred_element_type=jnp.float32,
            )
            return _

        lax.fori_loop(0, N_DEV, gemm_chunk, None)

        bar = pltpu.get_barrier_semaphore()
        for nbr in (left, right):
            pl.semaphore_signal(
                bar, inc=1, device_id=(nbr,),
                device_id_type=pl.DeviceIdType.MESH,
            )
        pl.semaphore_wait(bar, 2)

        stage_a[0] = out_ref[pl.ds(rho * ch, ch), :n2].astype(jnp.bfloat16)
        stage_b[0] = out_ref[pl.ds(rho * ch, ch), n2:].astype(jnp.bfloat16)
        desc_a(stage_a.at[0], rs_comm_a.at[0], 0).start()
        desc_b(stage_b.at[0], rs_comm_b.at[0], 0).start()

        def rs_hop(s, _):
            par = jnp.mod(s, 2)
            nxt = jnp.mod(s + 1, 2)
            ra = jnp.mod(rho - s - 1, N_DEV)
            rb = jnp.mod(rho + s + 1, N_DEV)

            desc_a(stage_a.at[par], rs_comm_a.at[par], par).wait_recv()
            desc_b(stage_b.at[par], rs_comm_b.at[par], par).wait_recv()

            sum_a = (out_ref[pl.ds(ra * ch, ch), :n2]
                     + rs_comm_a[par].astype(jnp.float32))
            out_ref[pl.ds(ra * ch, ch), :n2] = sum_a
            sum_b = (out_ref[pl.ds(rb * ch, ch), n2:]
                     + rs_comm_b[par].astype(jnp.float32))
            out_ref[pl.ds(rb * ch, ch), n2:] = sum_b

            @pl.when(s < N_DEV - 2)
            def _():
                @pl.when(s >= 1)
                def _():
                    desc_a(stage_a.at[nxt], rs_comm_a.at[nxt],
                           nxt).wait_send()
                    desc_b(stage_b.at[nxt], rs_comm_b.at[nxt],
                           nxt).wait_send()
                    pl.semaphore_wait(credit_a, 1)
                    pl.semaphore_wait(credit_b, 1)
                stage_a[nxt] = sum_a.astype(jnp.bfloat16)
                stage_b[nxt] = sum_b.astype(jnp.bfloat16)
                desc_a(stage_a.at[nxt], rs_comm_a.at[nxt], nxt).start()
                desc_b(stage_b.at[nxt], rs_comm_b.at[nxt], nxt).start()

            @pl.when(s < N_DEV - 3)
            def _():
                pl.semaphore_signal(
                    credit_a, inc=1, device_id=(left,),
                    device_id_type=pl.DeviceIdType.MESH,
                )
                pl.semaphore_signal(
                    credit_b, inc=1, device_id=(right,),
                    device_id_type=pl.DeviceIdType.MESH,
                )
            return _

        lax.fori_loop(0, N_DEV - 1, rs_hop, None)

        desc_a(stage_a.at[1], rs_comm_a.at[1], 1).wait_send()
        desc_b(stage_b.at[1], rs_comm_b.at[1], 1).wait_send()
        desc_a(stage_a.at[0], rs_comm_a.at[0], 0).wait_send()
        desc_b(stage_b.at[0], rs_comm_b.at[0], 0).wait_send()

        own_a = jnp.mod(rho + 1, N_DEV)
        own_b = jnp.mod(rho - 1, N_DEV)
        scale = sx_ref[0] * sw_ref[0]
        rows_a = pl.ds(own_a * ch, ch)
        rows_b = pl.ds(own_b * ch, ch)
        out_ref[rows_a, :n2] = jnp.maximum(out_ref[rows_a, :n2] * scale, 0.0)
        out_ref[rows_b, n2:] = jnp.maximum(out_ref[rows_b, n2:] * scale, 0.0)
        stage_a[0] = out_ref[rows_a, :n2].astype(jnp.bfloat16)
        stage_b[0] = out_ref[rows_b, n2:].astype(jnp.bfloat16)

        desc_a(stage_a.at[0], ag_comm_a.at[0], 0).start()
        desc_b(stage_b.at[0], ag_comm_b.at[0], 0).start()

        def ag_hop(s, _):
            par = jnp.mod(s, 2)
            nxt = jnp.mod(s + 1, 2)
            ra = jnp.mod(rho - s, N_DEV)
            rb = jnp.mod(rho + s, N_DEV)

            desc_a(ag_comm_a.at[par], ag_comm_a.at[par], par).wait_recv()
            desc_b(ag_comm_b.at[par], ag_comm_b.at[par], par).wait_recv()

            @pl.when(s < N_DEV - 2)
            def _():
                @pl.when(s >= 1)
                def _():
                    pl.semaphore_wait(credit_a, 1)
                    pl.semaphore_wait(credit_b, 1)
                desc_a(ag_comm_a.at[par], ag_comm_a.at[nxt], nxt).start()
                desc_b(ag_comm_b.at[par], ag_comm_b.at[nxt], nxt).start()

            out_ref[pl.ds(ra * ch, ch), :n2] = (
                ag_comm_a[par].astype(jnp.float32))
            out_ref[pl.ds(rb * ch, ch), n2:] = (
                ag_comm_b[par].astype(jnp.float32))

            @pl.when(s < N_DEV - 2)
            def _():
                @pl.when(s == 0)
                def _():
                    desc_a(stage_a.at[0], ag_comm_a.at[0], 0).wait_send()
                    desc_b(stage_b.at[0], ag_comm_b.at[0], 0).wait_send()
                desc_a(ag_comm_a.at[par], ag_comm_a.at[nxt], nxt).wait_send()
                desc_b(ag_comm_b.at[par], ag_comm_b.at[nxt], nxt).wait_send()

            @pl.when(s < N_DEV - 3)
            def _():
                pl.semaphore_signal(
                    credit_a, inc=1, device_id=(left,),
                    device_id_type=pl.DeviceIdType.MESH,
                )
                pl.semaphore_signal(
                    credit_b, inc=1, device_id=(right,),
                    device_id_type=pl.DeviceIdType.MESH,
                )
            return _

        lax.fori_loop(0, N_DEV - 1, ag_hop, None)

    mesh_idx = lax.axis_index("i")
    rho = jnp.take(jnp.asarray(_CYCLE_INV), mesh_idx)
    cyc = jnp.asarray(_CYCLE)
    left = jnp.take(cyc, jnp.mod(rho - 1, N_DEV))
    right = jnp.take(cyc, jnp.mod(rho + 1, N_DEV))
    ids = jnp.stack([rho, left, right]).astype(jnp.int32)

    return pl.pallas_call(
        body,
        out_shape=jax.ShapeDtypeStruct((m, n), jnp.float32),
        in_specs=[
            pl.BlockSpec(memory_space=pltpu.VMEM),
            pl.BlockSpec(memory_space=pltpu.VMEM),
            pl.BlockSpec(memory_space=pltpu.SMEM),
            pl.BlockSpec(memory_space=pltpu.SMEM),
            pl.BlockSpec(memory_space=pltpu.SMEM),
        ],
        out_specs=pl.BlockSpec(memory_space=pltpu.VMEM),
        scratch_shapes=[
            pltpu.VMEM((2, ch, n2), jnp.bfloat16),
            pltpu.VMEM((2, ch, n2), jnp.bfloat16),
            pltpu.VMEM((2, ch, n2), jnp.bfloat16),
            pltpu.VMEM((2, ch, n2), jnp.bfloat16),
            pltpu.VMEM((2, ch, n2), jnp.bfloat16),
            pltpu.VMEM((2, ch, n2), jnp.bfloat16),
            pltpu.SemaphoreType.DMA((2,)),
            pltpu.SemaphoreType.DMA((2,)),
            pltpu.SemaphoreType.DMA((2,)),
            pltpu.SemaphoreType.DMA((2,)),
            pltpu.SemaphoreType.REGULAR,
            pltpu.SemaphoreType.REGULAR,
        ],
        compiler_params=pltpu.CompilerParams(
            collective_id=0,
            vmem_limit_bytes=60 * 1024 * 1024,
        ),
    )(x, w_mat, scale_x, scale_w, ids)
